# Initial kernel scaffold; baseline (speedup 1.0000x reference)
import jax
import jax.numpy as jnp
from jax.experimental import pallas as pl


def kernel(entity_emb, A1, B1, W1, att_l1, att_r1, rhb1, b1, A2, B2, W2, att_l2, att_r2, rhb2, b2, Wres, bres, gamma, beta, edge_index, edge_type):
    raise NotImplementedError("write your pallas kernel here")


# speedup vs baseline: 13.7793x; 13.7793x over previous
import functools
import jax
import jax.numpy as jnp
from jax.experimental import pallas as pl

_N_REL = 8
_RANK = 8
_SCALE = 1.0
_HEADS = 4
_HID = 128
_OUT = 256
_EMB = 128


def _mm_bias_kernel(x_ref, w_ref, b_ref, o_ref, *, act):
    y = jnp.dot(x_ref[...], w_ref[...], preferred_element_type=jnp.float32)
    y = y + b_ref[...]
    if act == "elu":
        y = jnp.where(y > 0, y, jnp.exp(jnp.minimum(y, 0.0)) - 1.0)
    o_ref[...] = y


def _mm_bias(x, w, b, act, m_tile):
    M, K = x.shape
    _, N = w.shape
    return pl.pallas_call(
        functools.partial(_mm_bias_kernel, act=act),
        grid=(M // m_tile,),
        in_specs=[
            pl.BlockSpec((m_tile, K), lambda i: (i, 0)),
            pl.BlockSpec((K, N), lambda i: (0, 0)),
            pl.BlockSpec((1, N), lambda i: (0, 0)),
        ],
        out_specs=pl.BlockSpec((m_tile, N), lambda i: (i, 0)),
        out_shape=jax.ShapeDtypeStruct((M, N), jnp.float32),
    )(x, w, b.reshape(1, N))


def _mm_ln_kernel(x_ref, w_ref, b_ref, g_ref, bt_ref, o_ref):
    y = jnp.dot(x_ref[...], w_ref[...], preferred_element_type=jnp.float32)
    y = y + b_ref[...]
    mu = jnp.mean(y, axis=-1, keepdims=True)
    d = y - mu
    var = jnp.mean(d * d, axis=-1, keepdims=True)
    o_ref[...] = g_ref[...] * d * jax.lax.rsqrt(var + 1e-5) + bt_ref[...]


def _mm_ln(x, w, b, gamma, beta, m_tile):
    M, K = x.shape
    _, N = w.shape
    return pl.pallas_call(
        _mm_ln_kernel,
        grid=(M // m_tile,),
        in_specs=[
            pl.BlockSpec((m_tile, K), lambda i: (i, 0)),
            pl.BlockSpec((K, N), lambda i: (0, 0)),
            pl.BlockSpec((1, N), lambda i: (0, 0)),
            pl.BlockSpec((1, N), lambda i: (0, 0)),
            pl.BlockSpec((1, N), lambda i: (0, 0)),
        ],
        out_specs=pl.BlockSpec((m_tile, N), lambda i: (i, 0)),
        out_shape=jax.ShapeDtypeStruct((M, N), jnp.float32),
    )(x, w, b.reshape(1, N), gamma.reshape(1, N), beta.reshape(1, N))


def _edge_softmax(li_e, rj_e, rhb_e, key, n_seg):
    logit = li_e + rj_e + rhb_e
    a = jnp.where(logit > 0, logit, 0.2 * logit)
    amax = jax.ops.segment_max(a, key, num_segments=n_seg)
    amax = jnp.where(jnp.isfinite(amax), amax, 0.0)
    ea = jnp.exp(a - amax[key])
    den = jax.ops.segment_sum(ea, key, num_segments=n_seg)
    return ea / (den[key] + 1e-16)


def _gat_layer(x, i_idx, j_idx, t_idx, key, M1, rhb, heads):
    N = x.shape[0]
    E = i_idx.shape[0]
    nod = _mm_bias(x, M1, jnp.zeros((M1.shape[1],), jnp.float32), None, 1000)
    li = nod[:, :heads]
    rj = nod[:, heads:heads + _N_REL * heads].reshape(N * _N_REL, heads)
    P = nod[:, heads + _N_REL * heads:heads + _N_REL * heads + _N_REL * _RANK]
    P = P.reshape(N * _N_REL, _RANK)

    jt = j_idx * _N_REL + t_idx
    w = _edge_softmax(li[i_idx], rj[jt], rhb[t_idx], key, N * _N_REL)

    xg = x[j_idx]
    G = []
    for h in range(heads):
        G.append(jax.ops.segment_sum(xg * w[:, h:h + 1], i_idx, num_segments=N))
    G = jnp.stack(G, axis=1)

    Pe = P[jt]
    Qe = (w[:, :, None] * Pe[:, None, :]).reshape(E, heads * _RANK)
    Q = jax.ops.segment_sum(Qe, key, num_segments=N * _N_REL)
    Q = Q.reshape(N, _N_REL, heads, _RANK).transpose(0, 2, 1, 3)
    return G, Q


def _fold_attention(W, att_l, att_r, A, B, heads, out_dim, in_dim):
    Wr = W.reshape(heads, out_dim, in_dim)
    wl = jnp.einsum('hqc,hq->ch', Wr, att_l[0].reshape(heads, out_dim))
    wrb = jnp.einsum('hqc,hq->ch', Wr, att_r[0].reshape(heads, out_dim))
    Am = A.reshape(_N_REL, in_dim, _RANK)
    Bm = B.reshape(_N_REL, in_dim, _RANK)
    WR = wrb[None] + _SCALE * jnp.einsum('rcp,rdp,dh->rch', Bm, Am, wrb)
    M1 = jnp.concatenate([
        wl,
        WR.transpose(1, 0, 2).reshape(in_dim, _N_REL * heads),
        Bm.transpose(1, 0, 2).reshape(in_dim, _N_REL * _RANK),
    ], axis=1)
    kh = in_dim + _N_REL * _RANK
    Wbd = jnp.zeros((heads * kh, heads * out_dim), jnp.float32)
    for h in range(heads):
        WhT = Wr[h].T
        Wbd = Wbd.at[h * kh:h * kh + in_dim, h * out_dim:(h + 1) * out_dim].set(WhT)
        AW = jnp.einsum('rcp,cq->rpq', Am, WhT).reshape(_N_REL * _RANK, out_dim)
        Wbd = Wbd.at[h * kh + in_dim:(h + 1) * kh,
                     h * out_dim:(h + 1) * out_dim].set(_SCALE * AW)
    return M1, Wbd


def kernel(entity_emb, A1, B1, W1, att_l1, att_r1, rhb1, b1, A2, B2, W2,
           att_l2, att_r2, rhb2, b2, Wres, bres, gamma, beta, edge_index,
           edge_type):
    x0 = entity_emb
    N = x0.shape[0]
    j_idx = edge_index[0]
    i_idx = edge_index[1]
    t_idx = edge_type
    key = i_idx * _N_REL + t_idx

    M1, Wbd1 = _fold_attention(W1, att_l1, att_r1, A1, B1, _HEADS, _HID, _EMB)
    G1, Q1 = _gat_layer(x0, i_idx, j_idx, t_idx, key, M1, rhb1, _HEADS)
    kh1 = _EMB + _N_REL * _RANK
    Xc1 = jnp.concatenate([G1, Q1.reshape(N, _HEADS, _N_REL * _RANK)], axis=-1)
    Xc1 = Xc1.reshape(N, _HEADS * kh1)
    x1 = _mm_bias(Xc1, Wbd1, b1, "elu", 1000)

    in2 = _HEADS * _HID
    M2, Wbd2 = _fold_attention(W2, att_l2, att_r2, A2, B2, 1, _OUT, in2)
    G2, Q2 = _gat_layer(x1, i_idx, j_idx, t_idx, key, M2, rhb2, 1)
    Xc2 = jnp.concatenate([
        G2.reshape(N, in2),
        Q2.reshape(N, _N_REL * _RANK),
        x0,
    ], axis=-1)
    Wcat = jnp.concatenate([
        Wbd2,
        Wres.T,
    ], axis=0)
    return _mm_ln(Xc2, Wcat, b2 + bres, gamma, beta, 1000)


# speedup vs baseline: 14.1701x; 1.0284x over previous
import functools
import jax
import jax.numpy as jnp
from jax.experimental import pallas as pl

_N_REL = 8
_RANK = 8
_SCALE = 1.0
_HEADS = 4
_HID = 128
_OUT = 256
_EMB = 128


def _mm_bias_kernel(x_ref, w_ref, b_ref, o_ref, *, act):
    y = jnp.dot(x_ref[...], w_ref[...], preferred_element_type=jnp.float32)
    y = y + b_ref[...]
    if act == "elu":
        y = jnp.where(y > 0, y, jnp.exp(jnp.minimum(y, 0.0)) - 1.0)
    o_ref[...] = y


def _mm_bias(x, w, b, act, m_tile):
    M, K = x.shape
    _, N = w.shape
    return pl.pallas_call(
        functools.partial(_mm_bias_kernel, act=act),
        grid=(M // m_tile,),
        in_specs=[
            pl.BlockSpec((m_tile, K), lambda i: (i, 0)),
            pl.BlockSpec((K, N), lambda i: (0, 0)),
            pl.BlockSpec((1, N), lambda i: (0, 0)),
        ],
        out_specs=pl.BlockSpec((m_tile, N), lambda i: (i, 0)),
        out_shape=jax.ShapeDtypeStruct((M, N), jnp.float32),
    )(x, w, b.reshape(1, N))


def _mm_ln_kernel(x_ref, w_ref, b_ref, g_ref, bt_ref, o_ref):
    y = jnp.dot(x_ref[...], w_ref[...], preferred_element_type=jnp.float32)
    y = y + b_ref[...]
    mu = jnp.mean(y, axis=-1, keepdims=True)
    d = y - mu
    var = jnp.mean(d * d, axis=-1, keepdims=True)
    o_ref[...] = g_ref[...] * d * jax.lax.rsqrt(var + 1e-5) + bt_ref[...]


def _mm_ln(x, w, b, gamma, beta, m_tile):
    M, K = x.shape
    _, N = w.shape
    return pl.pallas_call(
        _mm_ln_kernel,
        grid=(M // m_tile,),
        in_specs=[
            pl.BlockSpec((m_tile, K), lambda i: (i, 0)),
            pl.BlockSpec((K, N), lambda i: (0, 0)),
            pl.BlockSpec((1, N), lambda i: (0, 0)),
            pl.BlockSpec((1, N), lambda i: (0, 0)),
            pl.BlockSpec((1, N), lambda i: (0, 0)),
        ],
        out_specs=pl.BlockSpec((m_tile, N), lambda i: (i, 0)),
        out_shape=jax.ShapeDtypeStruct((M, N), jnp.float32),
    )(x, w, b.reshape(1, N), gamma.reshape(1, N), beta.reshape(1, N))


def _edge_elem(kernel_fn, outs, *ins, e_tile=4000):
    E = ins[0].shape[0]
    return pl.pallas_call(
        kernel_fn,
        grid=(E // e_tile,),
        in_specs=[pl.BlockSpec((e_tile, a.shape[1]), lambda i: (i, 0))
                  for a in ins],
        out_specs=[pl.BlockSpec((e_tile, s[1]), lambda i: (i, 0))
                   for s in outs],
        out_shape=[jax.ShapeDtypeStruct((E, s[1]), jnp.float32) for s in outs],
    )(*ins)


def _logit_kernel(li_ref, rj_ref, rhb_ref, a_ref):
    logit = li_ref[...] + rj_ref[...] + rhb_ref[...]
    a_ref[...] = jnp.where(logit > 0, logit, 0.2 * logit)


def _exp_kernel(a_ref, amax_ref, ea_ref):
    ea_ref[...] = jnp.exp(a_ref[...] - amax_ref[...])


def _payload_kernel(ea_ref, den_ref, xg_ref, pe_ref, wx_ref, qe_ref, *, heads):
    w = ea_ref[...] / (den_ref[...] + 1e-16)
    xg = xg_ref[...]
    pe = pe_ref[...]
    wx_ref[...] = jnp.concatenate(
        [xg * w[:, h:h + 1] for h in range(heads)], axis=1)
    qe_ref[...] = jnp.concatenate(
        [pe * w[:, h:h + 1] for h in range(heads)], axis=1)


def _edge_softmax(li_e, rj_e, rhb_e, key, n_seg):
    (a,) = _edge_elem(_logit_kernel, [(0, li_e.shape[1])], li_e, rj_e, rhb_e)
    amax = jax.ops.segment_max(a, key, num_segments=n_seg)
    amax = jnp.where(jnp.isfinite(amax), amax, 0.0)
    (ea,) = _edge_elem(_exp_kernel, [(0, a.shape[1])], a, amax[key])
    den = jax.ops.segment_sum(ea, key, num_segments=n_seg)
    return ea, den[key]


def _gat_layer(x, i_idx, j_idx, t_idx, key, M1, rhb, heads):
    N = x.shape[0]
    E = i_idx.shape[0]
    nod = _mm_bias(x, M1, jnp.zeros((M1.shape[1],), jnp.float32), None, 1000)
    li = nod[:, :heads]
    rj = nod[:, heads:heads + _N_REL * heads].reshape(N * _N_REL, heads)
    P = nod[:, heads + _N_REL * heads:heads + _N_REL * heads + _N_REL * _RANK]
    P = P.reshape(N * _N_REL, _RANK)

    jt = j_idx * _N_REL + t_idx
    ea, den_g = _edge_softmax(li[i_idx], rj[jt], rhb[t_idx], key, N * _N_REL)

    xg = x[j_idx]
    Pe = P[jt]
    C = xg.shape[1]
    wx, Qe = _edge_elem(
        functools.partial(_payload_kernel, heads=heads),
        [(0, heads * C), (0, heads * _RANK)], ea, den_g, xg, Pe)
    G = jax.ops.segment_sum(wx, i_idx, num_segments=N)
    G = G.reshape(N, heads, C)
    Q = jax.ops.segment_sum(Qe, key, num_segments=N * _N_REL)
    Q = Q.reshape(N, _N_REL, heads, _RANK).transpose(0, 2, 1, 3)
    return G, Q


def _fold_attention(W, att_l, att_r, A, B, heads, out_dim, in_dim):
    Wr = W.reshape(heads, out_dim, in_dim)
    wl = jnp.einsum('hqc,hq->ch', Wr, att_l[0].reshape(heads, out_dim))
    wrb = jnp.einsum('hqc,hq->ch', Wr, att_r[0].reshape(heads, out_dim))
    Am = A.reshape(_N_REL, in_dim, _RANK)
    Bm = B.reshape(_N_REL, in_dim, _RANK)
    WR = wrb[None] + _SCALE * jnp.einsum('rcp,rdp,dh->rch', Bm, Am, wrb)
    M1 = jnp.concatenate([
        wl,
        WR.transpose(1, 0, 2).reshape(in_dim, _N_REL * heads),
        Bm.transpose(1, 0, 2).reshape(in_dim, _N_REL * _RANK),
    ], axis=1)
    kh = in_dim + _N_REL * _RANK
    Wbd = jnp.zeros((heads * kh, heads * out_dim), jnp.float32)
    for h in range(heads):
        WhT = Wr[h].T
        Wbd = Wbd.at[h * kh:h * kh + in_dim, h * out_dim:(h + 1) * out_dim].set(WhT)
        AW = jnp.einsum('rcp,cq->rpq', Am, WhT).reshape(_N_REL * _RANK, out_dim)
        Wbd = Wbd.at[h * kh + in_dim:(h + 1) * kh,
                     h * out_dim:(h + 1) * out_dim].set(_SCALE * AW)
    return M1, Wbd


def kernel(entity_emb, A1, B1, W1, att_l1, att_r1, rhb1, b1, A2, B2, W2,
           att_l2, att_r2, rhb2, b2, Wres, bres, gamma, beta, edge_index,
           edge_type):
    x0 = entity_emb
    N = x0.shape[0]
    j_idx = edge_index[0]
    i_idx = edge_index[1]
    t_idx = edge_type
    key = i_idx * _N_REL + t_idx

    M1, Wbd1 = _fold_attention(W1, att_l1, att_r1, A1, B1, _HEADS, _HID, _EMB)
    G1, Q1 = _gat_layer(x0, i_idx, j_idx, t_idx, key, M1, rhb1, _HEADS)
    kh1 = _EMB + _N_REL * _RANK
    Xc1 = jnp.concatenate([G1, Q1.reshape(N, _HEADS, _N_REL * _RANK)], axis=-1)
    Xc1 = Xc1.reshape(N, _HEADS * kh1)
    x1 = _mm_bias(Xc1, Wbd1, b1, "elu", 1000)

    in2 = _HEADS * _HID
    M2, Wbd2 = _fold_attention(W2, att_l2, att_r2, A2, B2, 1, _OUT, in2)
    G2, Q2 = _gat_layer(x1, i_idx, j_idx, t_idx, key, M2, rhb2, 1)
    Xc2 = jnp.concatenate([
        G2.reshape(N, in2),
        Q2.reshape(N, _N_REL * _RANK),
        x0,
    ], axis=-1)
    Wcat = jnp.concatenate([
        Wbd2,
        Wres.T,
    ], axis=0)
    return _mm_ln(Xc2, Wcat, b2 + bres, gamma, beta, 1000)


# speedup vs baseline: 15.0698x; 1.0635x over previous
import functools
import jax
import jax.numpy as jnp
from jax.experimental import pallas as pl

_N_REL = 8
_RANK = 8
_SCALE = 1.0
_HEADS = 4
_HID = 128
_OUT = 256
_EMB = 128


def _mm_bias_kernel(x_ref, w_ref, b_ref, o_ref, *, act):
    y = jnp.dot(x_ref[...], w_ref[...], preferred_element_type=jnp.float32)
    y = y + b_ref[...]
    if act == "elu":
        y = jnp.where(y > 0, y, jnp.exp(jnp.minimum(y, 0.0)) - 1.0)
    o_ref[...] = y


def _mm_bias(x, w, b, act, m_tile):
    M, K = x.shape
    _, N = w.shape
    return pl.pallas_call(
        functools.partial(_mm_bias_kernel, act=act),
        grid=(M // m_tile,),
        in_specs=[
            pl.BlockSpec((m_tile, K), lambda i: (i, 0)),
            pl.BlockSpec((K, N), lambda i: (0, 0)),
            pl.BlockSpec((1, N), lambda i: (0, 0)),
        ],
        out_specs=pl.BlockSpec((m_tile, N), lambda i: (i, 0)),
        out_shape=jax.ShapeDtypeStruct((M, N), jnp.float32),
    )(x, w, b.reshape(1, N))


def _mm_ln_kernel(x_ref, w_ref, b_ref, g_ref, bt_ref, o_ref):
    y = jnp.dot(x_ref[...], w_ref[...], preferred_element_type=jnp.float32)
    y = y + b_ref[...]
    mu = jnp.mean(y, axis=-1, keepdims=True)
    d = y - mu
    var = jnp.mean(d * d, axis=-1, keepdims=True)
    o_ref[...] = g_ref[...] * d * jax.lax.rsqrt(var + 1e-5) + bt_ref[...]


def _mm_ln(x, w, b, gamma, beta, m_tile):
    M, K = x.shape
    _, N = w.shape
    return pl.pallas_call(
        _mm_ln_kernel,
        grid=(M // m_tile,),
        in_specs=[
            pl.BlockSpec((m_tile, K), lambda i: (i, 0)),
            pl.BlockSpec((K, N), lambda i: (0, 0)),
            pl.BlockSpec((1, N), lambda i: (0, 0)),
            pl.BlockSpec((1, N), lambda i: (0, 0)),
            pl.BlockSpec((1, N), lambda i: (0, 0)),
        ],
        out_specs=pl.BlockSpec((m_tile, N), lambda i: (i, 0)),
        out_shape=jax.ShapeDtypeStruct((M, N), jnp.float32),
    )(x, w, b.reshape(1, N), gamma.reshape(1, N), beta.reshape(1, N))


def _edge_elem(kernel_fn, outs, *ins, e_tile=4000):
    E = ins[0].shape[0]
    return pl.pallas_call(
        kernel_fn,
        grid=(E // e_tile,),
        in_specs=[pl.BlockSpec((e_tile, a.shape[1]), lambda i: (i, 0))
                  for a in ins],
        out_specs=[pl.BlockSpec((e_tile, s[1]), lambda i: (i, 0))
                   for s in outs],
        out_shape=[jax.ShapeDtypeStruct((E, s[1]), jnp.float32) for s in outs],
    )(*ins)


def _logit_kernel(li_ref, rj_ref, rhb_ref, a_ref):
    logit = li_ref[...] + rj_ref[...] + rhb_ref[...]
    a_ref[...] = jnp.where(logit > 0, logit, 0.2 * logit)


def _exp_kernel(a_ref, amax_ref, ea_ref):
    ea_ref[...] = jnp.exp(a_ref[...] - amax_ref[...])


def _payload_kernel(ea_ref, den_ref, xg_ref, pe_ref, wx_ref, qe_ref, *, heads):
    w = ea_ref[...] / (den_ref[...] + 1e-16)
    xg = xg_ref[...]
    pe = pe_ref[...]
    wx_ref[...] = jnp.concatenate(
        [xg * w[:, h:h + 1] for h in range(heads)], axis=1)
    qe_ref[...] = jnp.concatenate(
        [pe * w[:, h:h + 1] for h in range(heads)], axis=1)


def _edge_softmax(li_e, rj_e, rhb_e, key, n_seg):
    (a,) = _edge_elem(_logit_kernel, [(0, li_e.shape[1])], li_e, rj_e, rhb_e)
    amax = jax.ops.segment_max(a, key, num_segments=n_seg)
    amax = jnp.where(jnp.isfinite(amax), amax, 0.0)
    (ea,) = _edge_elem(_exp_kernel, [(0, a.shape[1])], a, amax[key])
    den = jax.ops.segment_sum(ea, key, num_segments=n_seg)
    return ea, den[key]


def _gat_layer(x, i_idx, j_idx, t_idx, key, M1, rhb, heads, proj=None):
    N = x.shape[0]
    E = i_idx.shape[0]
    base = heads + _N_REL * heads + _N_REL * _RANK
    if proj is not None:
        M1 = jnp.concatenate([M1, proj], axis=1)
    nod = _mm_bias(x, M1, jnp.zeros((M1.shape[1],), jnp.float32), None, 1000)
    li = nod[:, :heads]
    rj = nod[:, heads:heads + _N_REL * heads].reshape(N * _N_REL, heads)
    P = nod[:, heads + _N_REL * heads:base]
    P = P.reshape(N * _N_REL, _RANK)
    xprop = x if proj is None else nod[:, base:]

    jt = j_idx * _N_REL + t_idx
    ea, den_g = _edge_softmax(li[i_idx], rj[jt], rhb[t_idx], key, N * _N_REL)

    xg = xprop[j_idx]
    Pe = P[jt]
    C = xg.shape[1]
    wx, Qe = _edge_elem(
        functools.partial(_payload_kernel, heads=heads),
        [(0, heads * C), (0, heads * _RANK)], ea, den_g, xg, Pe)
    G = jax.ops.segment_sum(wx, i_idx, num_segments=N)
    G = G.reshape(N, heads, C)
    Q = jax.ops.segment_sum(Qe, key, num_segments=N * _N_REL)
    Q = Q.reshape(N, _N_REL, heads, _RANK).transpose(0, 2, 1, 3)
    return G, Q


def _fold_attention(W, att_l, att_r, A, B, heads, out_dim, in_dim):
    Wr = W.reshape(heads, out_dim, in_dim)
    wl = jnp.einsum('hqc,hq->ch', Wr, att_l[0].reshape(heads, out_dim))
    wrb = jnp.einsum('hqc,hq->ch', Wr, att_r[0].reshape(heads, out_dim))
    Am = A.reshape(_N_REL, in_dim, _RANK)
    Bm = B.reshape(_N_REL, in_dim, _RANK)
    WR = wrb[None] + _SCALE * jnp.einsum('rcp,rdp,dh->rch', Bm, Am, wrb)
    M1 = jnp.concatenate([
        wl,
        WR.transpose(1, 0, 2).reshape(in_dim, _N_REL * heads),
        Bm.transpose(1, 0, 2).reshape(in_dim, _N_REL * _RANK),
    ], axis=1)
    kh = in_dim + _N_REL * _RANK
    Wbd = jnp.zeros((heads * kh, heads * out_dim), jnp.float32)
    for h in range(heads):
        WhT = Wr[h].T
        Wbd = Wbd.at[h * kh:h * kh + in_dim, h * out_dim:(h + 1) * out_dim].set(WhT)
        AW = jnp.einsum('rcp,cq->rpq', Am, WhT).reshape(_N_REL * _RANK, out_dim)
        Wbd = Wbd.at[h * kh + in_dim:(h + 1) * kh,
                     h * out_dim:(h + 1) * out_dim].set(_SCALE * AW)
    return M1, Wbd


def kernel(entity_emb, A1, B1, W1, att_l1, att_r1, rhb1, b1, A2, B2, W2,
           att_l2, att_r2, rhb2, b2, Wres, bres, gamma, beta, edge_index,
           edge_type):
    x0 = entity_emb
    N = x0.shape[0]
    j_idx = edge_index[0]
    i_idx = edge_index[1]
    t_idx = edge_type
    key = i_idx * _N_REL + t_idx

    M1, Wbd1 = _fold_attention(W1, att_l1, att_r1, A1, B1, _HEADS, _HID, _EMB)
    G1, Q1 = _gat_layer(x0, i_idx, j_idx, t_idx, key, M1, rhb1, _HEADS)
    kh1 = _EMB + _N_REL * _RANK
    Xc1 = jnp.concatenate([G1, Q1.reshape(N, _HEADS, _N_REL * _RANK)], axis=-1)
    Xc1 = Xc1.reshape(N, _HEADS * kh1)
    x1 = _mm_bias(Xc1, Wbd1, b1, "elu", 1000)

    in2 = _HEADS * _HID
    M2, Wbd2 = _fold_attention(W2, att_l2, att_r2, A2, B2, 1, _OUT, in2)
    G2, Q2 = _gat_layer(x1, i_idx, j_idx, t_idx, key, M2, rhb2, 1, proj=W2.T)
    Xc2 = jnp.concatenate([
        G2.reshape(N, _OUT),
        Q2.reshape(N, _N_REL * _RANK),
        x0,
    ], axis=-1)
    Wcat = jnp.concatenate([
        jnp.eye(_OUT, dtype=jnp.float32),
        Wbd2[in2:, :],
        Wres.T,
    ], axis=0)
    return _mm_ln(Xc2, Wcat, b2 + bres, gamma, beta, 1000)
